# baseline (speedup 1.0000x reference)
import functools

import jax
import jax.numpy as jnp
from jax import lax
from jax.experimental import pallas as pl
from jax.experimental.pallas import tpu as pltpu
from jax.experimental.pallas import tpu_sc as plsc

TILE_M = 1024
TILE_N = 2048


def _main_body(z_ref, cbt_ref, sim_ref, ids_ref, minv_ref, runmin_ref):
    j = pl.program_id(1)
    nj = pl.num_programs(1)
    z = z_ref[...]
    cbt = cbt_ref[...]
    dot = lax.dot_general(z, cbt, (((1,), (0,)), ((), ())),
                          preferred_element_type=jnp.float32)
    nz = jnp.sum(z * z, axis=1, keepdims=True)
    ne = jnp.sum(cbt * cbt, axis=0, keepdims=True)
    sim_ref[...] = lax.dot_general(z * lax.rsqrt(nz), cbt * lax.rsqrt(ne),
                                   (((1,), (0,)), ((), ())),
                                   preferred_element_type=jnp.float32)
    dist = (-2.0 * dot + nz) + ne
    lmin = jnp.min(dist, axis=1, keepdims=True)
    larg = (jnp.argmin(dist, axis=1).reshape(lmin.shape).astype(jnp.int32)
            + j * TILE_N)

    @pl.when(j == 0)
    def _():
        runmin_ref[...] = lmin
        ids_ref[...] = larg

    @pl.when(j > 0)
    def _():
        better = lmin < runmin_ref[...]
        runmin_ref[...] = jnp.where(better, lmin, runmin_ref[...])
        ids_ref[...] = jnp.where(better, larg, ids_ref[...])

    @pl.when(j == nj - 1)
    def _():
        minv_ref[...] = runmin_ref[...]


def _main_call(z2d, cbt, interpret=False):
    m, c = z2d.shape
    k = cbt.shape[1]
    grid = (m // TILE_M, k // TILE_N)
    return pl.pallas_call(
        _main_body,
        grid=grid,
        in_specs=[
            pl.BlockSpec((TILE_M, c), lambda i, j: (i, 0)),
            pl.BlockSpec((c, TILE_N), lambda i, j: (0, j)),
        ],
        out_specs=[
            pl.BlockSpec((TILE_M, TILE_N), lambda i, j: (i, j)),
            pl.BlockSpec((TILE_M, 1), lambda i, j: (i, 0)),
            pl.BlockSpec((TILE_M, 1), lambda i, j: (i, 0)),
        ],
        out_shape=[
            jax.ShapeDtypeStruct((m, k), jnp.float32),
            jax.ShapeDtypeStruct((m, 1), jnp.int32),
            jax.ShapeDtypeStruct((m, 1), jnp.float32),
        ],
        scratch_shapes=[pltpu.VMEM((TILE_M, 1), jnp.float32)],
        compiler_params=pltpu.CompilerParams(
            dimension_semantics=("parallel", "arbitrary")),
        interpret=interpret,
    )(z2d, cbt)


def _loss_body(minv_ref, out_ref):
    out_ref[0, 0] = jnp.sum(jnp.sqrt(jnp.maximum(minv_ref[...], 0.0)))


def _loss_call(minv2d, interpret=False):
    return pl.pallas_call(
        _loss_body,
        out_specs=pl.BlockSpec(memory_space=pltpu.SMEM),
        out_shape=jax.ShapeDtypeStruct((1, 1), jnp.float32),
        interpret=interpret,
    )(minv2d)


def _gather_call(codebook, ids):
    info = plsc.get_sparse_core_info()
    nc, ns = info.num_cores, info.num_subcores
    nw = nc * ns
    m = ids.shape[0]
    d = codebook.shape[1]
    b_per_w = m // nw
    chunk = 128
    nchunks = b_per_w // chunk
    mesh = plsc.VectorSubcoreMesh(core_axis_name="c", subcore_axis_name="s")

    @functools.partial(
        pl.kernel,
        mesh=mesh,
        out_type=jax.ShapeDtypeStruct((m, d), jnp.float32),
        scratch_types=[
            pltpu.VMEM((chunk,), jnp.int32),
            pltpu.VMEM((chunk, d), jnp.float32),
            pltpu.SemaphoreType.DMA,
        ],
    )
    def gk(table_hbm, idx_hbm, out_hbm, idx_v, rows_v, sem):
        wid = lax.axis_index("s") * nc + lax.axis_index("c")
        base = wid * b_per_w
        for ci in range(nchunks):
            off = base + ci * chunk
            pltpu.sync_copy(idx_hbm.at[pl.ds(off, chunk)], idx_v)
            pltpu.async_copy(table_hbm.at[idx_v], rows_v, sem).wait()
            pltpu.sync_copy(rows_v, out_hbm.at[pl.ds(off, chunk)])

    return gk(codebook, ids)


def kernel(z_e, codebook):
    b, t, c = z_e.shape
    k = codebook.shape[0]
    m = b * t
    z2d = z_e.reshape(m, c)
    cbt = codebook.T

    sim2d, ids2d, minv = _main_call(z2d, cbt)
    ids = ids2d.reshape(m)
    zq2d = _gather_call(codebook, ids)
    loss_sum = _loss_call(minv.reshape(128, m // 128))

    loss_mean = loss_sum[0, 0] / m
    loss_vq = loss_mean + loss_mean * 0.25

    return (zq2d.reshape(b, t, c),
            sim2d.reshape(b, t, k),
            ids.reshape(b, t),
            loss_vq)


# speedup vs baseline: 1.0778x; 1.0778x over previous
import functools

import jax
import jax.numpy as jnp
from jax import lax
from jax.experimental import pallas as pl
from jax.experimental.pallas import tpu as pltpu
from jax.experimental.pallas import tpu_sc as plsc

TILE_M = 2048
TILE_N = 1024


def _main_body(z_ref, cb_ref, sim_ref, ids_ref, loss_ref,
               runmin_ref, runids_ref):
    j = pl.program_id(1)
    nj = pl.num_programs(1)
    z = z_ref[...]
    cbt = cb_ref[...]
    dot = lax.dot_general(z, cbt, (((1,), (0,)), ((), ())),
                          preferred_element_type=jnp.float32)
    nz = jnp.sum(z * z, axis=1, keepdims=True)
    ne = jnp.sum(cbt * cbt, axis=0, keepdims=True)
    sim_ref[...] = dot * lax.rsqrt(nz) * lax.rsqrt(ne)
    dist = (-2.0 * dot + nz) + ne
    lmin = jnp.min(dist, axis=1, keepdims=True)
    col = lax.broadcasted_iota(jnp.int32, dist.shape, 1) + j * TILE_N
    larg = jnp.min(jnp.where(dist == lmin, col, jnp.int32(2 ** 30)),
                   axis=1, keepdims=True)

    @pl.when(j == 0)
    def _():
        runmin_ref[...] = lmin
        runids_ref[...] = larg

    @pl.when(j > 0)
    def _():
        better = lmin < runmin_ref[...]
        runmin_ref[...] = jnp.where(better, lmin, runmin_ref[...])
        runids_ref[...] = jnp.where(better, larg, runids_ref[...])

    @pl.when(j == nj - 1)
    def _():
        ids_ref[...] = runids_ref[...].T.reshape(1, 1, TILE_M)
        loss_ref[0, 0, 0] = jnp.sum(jnp.sqrt(jnp.maximum(runmin_ref[...], 0.0)))


def _main_call(z2d, cbt_in, interpret=False):
    m, c = z2d.shape
    k = cbt_in.shape[1]
    gm = m // TILE_M
    grid = (gm, k // TILE_N)
    return pl.pallas_call(
        _main_body,
        grid=grid,
        in_specs=[
            pl.BlockSpec((TILE_M, c), lambda i, j: (i, 0)),
            pl.BlockSpec((c, TILE_N), lambda i, j: (0, j)),
        ],
        out_specs=[
            pl.BlockSpec((TILE_M, TILE_N), lambda i, j: (i, j)),
            pl.BlockSpec((1, 1, TILE_M), lambda i, j: (i, 0, 0)),
            pl.BlockSpec((1, 1, 1), lambda i, j: (i, 0, 0),
                         memory_space=pltpu.SMEM),
        ],
        out_shape=[
            jax.ShapeDtypeStruct((m, k), jnp.float32),
            jax.ShapeDtypeStruct((gm, 1, TILE_M), jnp.int32),
            jax.ShapeDtypeStruct((gm, 1, 1), jnp.float32),
        ],
        scratch_shapes=[pltpu.VMEM((TILE_M, 1), jnp.float32),
                        pltpu.VMEM((TILE_M, 1), jnp.int32)],
        compiler_params=pltpu.CompilerParams(
            dimension_semantics=("parallel", "arbitrary")),
        interpret=interpret,
    )(z2d, cbt_in)


def _gather_call(codebook, ids):
    info = plsc.get_sparse_core_info()
    nc, ns = info.num_cores, info.num_subcores
    nw = nc * ns
    m = ids.shape[0]
    d = codebook.shape[1]
    b_per_w = m // nw
    chunk = 128
    nchunks = b_per_w // chunk
    mesh = plsc.VectorSubcoreMesh(core_axis_name="c", subcore_axis_name="s")

    @functools.partial(
        pl.kernel,
        mesh=mesh,
        out_type=jax.ShapeDtypeStruct((m, d), jnp.float32),
        scratch_types=[
            pltpu.VMEM((chunk,), jnp.int32),
            pltpu.VMEM((chunk, d), jnp.float32),
            pltpu.SemaphoreType.DMA,
        ],
    )
    def gk(table_hbm, idx_hbm, out_hbm, idx_v, rows_v, sem):
        wid = lax.axis_index("s") * nc + lax.axis_index("c")
        base = wid * b_per_w
        for ci in range(nchunks):
            off = base + ci * chunk
            pltpu.sync_copy(idx_hbm.at[pl.ds(off, chunk)], idx_v)
            pltpu.async_copy(table_hbm.at[idx_v], rows_v, sem).wait()
            pltpu.sync_copy(rows_v, out_hbm.at[pl.ds(off, chunk)])

    return gk(codebook, ids)


def kernel(z_e, codebook):
    b, t, c = z_e.shape
    k = codebook.shape[0]
    m = b * t
    z2d = z_e.reshape(m, c)

    sim2d, ids3d, loss_parts = _main_call(z2d, codebook.T)
    ids = ids3d.reshape(m)
    zq2d = _gather_call(codebook, ids)

    loss_mean = jnp.sum(loss_parts) / m
    loss_vq = loss_mean + loss_mean * 0.25

    return (zq2d.reshape(b, t, c),
            sim2d.reshape(b, t, k),
            ids.reshape(b, t),
            loss_vq)
